# baseline (speedup 1.0000x reference)
import functools

import jax
import jax.numpy as jnp
from jax import lax
from jax.experimental import pallas as pl
from jax.experimental.pallas import tpu as pltpu
from jax.experimental.pallas import tpu_sc as plsc

N_NODES = 10000
N_EDGES = 320000
NP = 10240
NC = 2
NS = 16
NW = NC * NS
CH = 80
K = 128
NB = 4
HD = 64
E_PAD = NW * K * CH
ROWS_PER_S = NP // NS


def _aggregate(hws, srcr, dstr):
    nh = len(hws)
    mesh = plsc.VectorSubcoreMesh(core_axis_name="c", subcore_axis_name="s")

    @functools.partial(
        pl.kernel,
        mesh=mesh,
        compiler_params=pltpu.CompilerParams(use_tc_tiling_on_sc=False),
        out_type=jax.ShapeDtypeStruct((NC, nh, NP, HD), jnp.float32),
        scratch_types=[
            pltpu.VMEM((K, CH), jnp.int32),
            pltpu.VMEM((K, CH), jnp.int32),
            pltpu.VMEM((NB, CH, HD), jnp.float32),
            pltpu.VMEM_SHARED((NP, HD), jnp.float32),
            pltpu.VMEM_SHARED((NP, HD), jnp.float32),
        ] + [pltpu.SemaphoreType.DMA] * NB,
    )
    def k(*args):
        hw_hbms = args[:nh]
        src_hbm, dst_hbm, out_hbm, src_v, dst_v, rows_v, tab_sh, acc_sh = args[nh:nh + 8]
        gsems = args[nh + 8:]
        c = lax.axis_index("c")
        s = lax.axis_index("s")
        w = s * NC + c

        pltpu.sync_copy(src_hbm.at[w], src_v)
        pltpu.sync_copy(dst_hbm.at[w], dst_v)

        for h, hw_hbm in enumerate(hw_hbms):
            def src_ref(m8):
                return hw_hbm if m8 == 7 else tab_sh

            def fire_gather(jj, b, m8):
                pltpu.async_copy(src_ref(m8).at[src_v.at[jj]], rows_v.at[b], gsems[b])

            def wait_gather(jj, b, m8):
                pltpu.make_async_copy(src_ref(m8).at[src_v.at[jj]], rows_v.at[b], gsems[b]).wait()

            def scatter_add(jj, b):
                pltpu.sync_copy(rows_v.at[b], acc_sh.at[dst_v.at[jj]], add=True)

            @pl.loop(0, CH)
            def _(r):
                for t in range(HD // 16):
                    rows_v[0, r, pl.ds(t * 16, 16)] = jnp.zeros((16,), jnp.float32)

            pltpu.sync_copy(
                hw_hbm.at[pl.ds(s * ROWS_PER_S, ROWS_PER_S)],
                tab_sh.at[pl.ds(s * ROWS_PER_S, ROWS_PER_S)],
            )

            @pl.loop(0, ROWS_PER_S // CH)
            def _(z):
                pltpu.sync_copy(rows_v.at[0], acc_sh.at[pl.ds(s * ROWS_PER_S + z * CH, CH)])

            if ROWS_PER_S % CH:
                pltpu.sync_copy(
                    rows_v.at[0, pl.ds(0, ROWS_PER_S % CH)],
                    acc_sh.at[pl.ds(s * ROWS_PER_S + (ROWS_PER_S // CH) * CH, ROWS_PER_S % CH)],
                )

            plsc.subcore_barrier()

            for i in range(NB):
                fire_gather(i, i, i)

            @pl.loop(0, K - 2 * NB, step=2 * NB)
            def _(j):
                for i in range(2 * NB):
                    jj = j + i
                    wait_gather(jj, i % NB, i)
                    scatter_add(jj, i % NB)
                    fire_gather(jj + NB, i % NB, (i + NB) % (2 * NB))

            for i in range(2 * NB):
                jj = K - 2 * NB + i
                wait_gather(jj, i % NB, i)
                scatter_add(jj, i % NB)
                if i < NB:
                    fire_gather(jj + NB, i % NB, (i + NB) % (2 * NB))

            plsc.subcore_barrier()

            pltpu.sync_copy(
                acc_sh.at[pl.ds(s * ROWS_PER_S, ROWS_PER_S)],
                out_hbm.at[c, h, pl.ds(s * ROWS_PER_S, ROWS_PER_S)],
            )

    return k(*hws, srcr, dstr)


_DOT = functools.partial(
    lax.dot_general,
    dimension_numbers=(((1,), (0,)), ((), ())),
    preferred_element_type=jnp.float32,
)

_R = 1024

_HALF_OUT = (
    [
        pl.BlockSpec((_R, HD), lambda i: (i, 0)),
        pl.BlockSpec((_R, HD), lambda i: (i, 0)),
    ],
    [
        jax.ShapeDtypeStruct((NP, HD), jnp.float32),
        jax.ShapeDtypeStruct((NP, HD), jnp.float32),
    ],
)


def _matmul_split(x, w):
    def body(x_ref, w_ref, o0_ref, o1_ref):
        xx = x_ref[...]
        o0_ref[...] = _DOT(xx, w_ref[:, :HD])
        o1_ref[...] = _DOT(xx, w_ref[:, HD:])

    return pl.pallas_call(
        body,
        grid=(NP // _R,),
        in_specs=[
            pl.BlockSpec((_R, 128), lambda i: (i, 0)),
            pl.BlockSpec((128, 128), lambda i: (0, 0)),
        ],
        out_specs=_HALF_OUT[0],
        out_shape=_HALF_OUT[1],
    )(x, w)


def _relu_matmul_split(p, b, w):
    def body(p_ref, b_ref, w_ref, o0_ref, o1_ref):
        a = jnp.concatenate(
            [
                jnp.maximum(p_ref[0, 0] + p_ref[1, 0] + b_ref[:, :HD], 0.0),
                jnp.maximum(p_ref[0, 1] + p_ref[1, 1] + b_ref[:, HD:], 0.0),
            ],
            axis=-1,
        )
        o0_ref[...] = _DOT(a, w_ref[:, :HD])
        o1_ref[...] = _DOT(a, w_ref[:, HD:])

    return pl.pallas_call(
        body,
        grid=(NP // _R,),
        in_specs=[
            pl.BlockSpec((NC, 2, _R, HD), lambda i: (0, 0, i, 0)),
            pl.BlockSpec((1, 128), lambda i: (0, 0)),
            pl.BlockSpec((128, 128), lambda i: (0, 0)),
        ],
        out_specs=_HALF_OUT[0],
        out_shape=_HALF_OUT[1],
    )(p, b, w)


def _relu_matmul_w3(p, b, w):
    def body(p_ref, b_ref, w_ref, o_ref):
        a = jnp.concatenate(
            [
                jnp.maximum(p_ref[0, 0] + p_ref[1, 0] + b_ref[:, :HD], 0.0),
                jnp.maximum(p_ref[0, 1] + p_ref[1, 1] + b_ref[:, HD:], 0.0),
            ],
            axis=-1,
        )
        o_ref[...] = _DOT(a, w_ref[...])

    return pl.pallas_call(
        body,
        grid=(NP // _R,),
        in_specs=[
            pl.BlockSpec((NC, 2, _R, HD), lambda i: (0, 0, i, 0)),
            pl.BlockSpec((1, 128), lambda i: (0, 0)),
            pl.BlockSpec((128, HD), lambda i: (0, 0)),
        ],
        out_specs=pl.BlockSpec((_R, HD), lambda i: (i, 0)),
        out_shape=jax.ShapeDtypeStruct((NP, HD), jnp.float32),
    )(p, b, w)


def _bias_log_softmax(q, b):
    def body(q_ref, b_ref, o_ref):
        t = q_ref[0, 0] + q_ref[1, 0] + b_ref[...]
        m = jnp.max(t, axis=-1, keepdims=True)
        e = jnp.exp(t - m)
        ssum = jnp.sum(e, axis=-1, keepdims=True)
        o_ref[...] = t - m - jnp.log(ssum)

    return pl.pallas_call(
        body,
        grid=(NP // _R,),
        in_specs=[
            pl.BlockSpec((NC, 1, _R, HD), lambda i: (0, 0, i, 0)),
            pl.BlockSpec((1, HD), lambda i: (0, 0)),
        ],
        out_specs=pl.BlockSpec((_R, HD), lambda i: (i, 0)),
        out_shape=jax.ShapeDtypeStruct((NP, HD), jnp.float32),
    )(q, b)


def kernel(x, edge_index, W1, b1, W2, b2, W3, b3):
    src = edge_index[0].astype(jnp.int32)
    dst = edge_index[1].astype(jnp.int32)
    npad = E_PAD - N_EDGES
    pad_iota = jnp.arange(npad, dtype=jnp.int32)
    srcr = jnp.concatenate([src, pad_iota % N_NODES]).reshape(NW, K, CH)
    dstr = jnp.concatenate([dst, N_NODES + pad_iota % (NP - N_NODES)]).reshape(NW, K, CH)

    xp = jnp.pad(x, ((0, NP - N_NODES), (0, 0)))
    b1r = b1.reshape(1, -1)
    b2r = b2.reshape(1, -1)
    b3r = b3.reshape(1, -1)

    h0, h1 = _matmul_split(xp, W1)
    p = _aggregate((h0, h1), srcr, dstr)
    h0, h1 = _relu_matmul_split(p, b1r, W2)
    p = _aggregate((h0, h1), srcr, dstr)
    h3 = _relu_matmul_w3(p, b2r, W3)
    q = _aggregate((h3,), srcr, dstr)
    out = _bias_log_softmax(q, b3r)
    return out[:N_NODES]


# speedup vs baseline: 1.0492x; 1.0492x over previous
import functools

import jax
import jax.numpy as jnp
from jax import lax
from jax.experimental import pallas as pl
from jax.experimental.pallas import tpu as pltpu
from jax.experimental.pallas import tpu_sc as plsc

N_NODES = 10000
N_EDGES = 320000
NP = 10240
NC = 2
NS = 16
NW = NC * NS
CH = 80
K = 128
NB = 4
HD = 64
E_PAD = NW * K * CH
ROWS_PER_S = NP // NS


def _aggregate(hws, srcr, dstr):
    nh = len(hws)
    mesh = plsc.VectorSubcoreMesh(core_axis_name="c", subcore_axis_name="s")

    @functools.partial(
        pl.kernel,
        mesh=mesh,
        compiler_params=pltpu.CompilerParams(use_tc_tiling_on_sc=False),
        out_type=jax.ShapeDtypeStruct((NC, nh, NP, HD), jnp.float32),
        scratch_types=[
            pltpu.VMEM((K, CH), jnp.int32),
            pltpu.VMEM((K, CH), jnp.int32),
            pltpu.VMEM((NB, CH, HD), jnp.float32),
            pltpu.VMEM((CH, HD), jnp.float32),
            pltpu.VMEM_SHARED((NP, HD), jnp.float32),
            pltpu.VMEM_SHARED((NP, HD), jnp.float32),
        ] + [pltpu.SemaphoreType.DMA] * (NB + 2),
    )
    def k(*args):
        hw_hbms = args[:nh]
        src_hbm, dst_hbm, out_hbm, src_v, dst_v, rows_v, zb_v, tab_sh, acc_sh = args[nh:nh + 9]
        gsems = args[nh + 9:nh + 9 + NB]
        stage_sem, wo_sem = args[nh + 9 + NB:]
        c = lax.axis_index("c")
        s = lax.axis_index("s")
        w = s * NC + c
        my_rows = pl.ds(s * ROWS_PER_S, ROWS_PER_S)

        h_src = pltpu.async_copy(src_hbm.at[w], src_v, gsems[0])
        h_dst = pltpu.async_copy(dst_hbm.at[w], dst_v, gsems[1])

        @pl.loop(0, CH)
        def _(r):
            for t in range(HD // 16):
                zb_v[r, pl.ds(t * 16, 16)] = jnp.zeros((16,), jnp.float32)

        def fire_stage(h):
            return pltpu.async_copy(hw_hbms[h].at[my_rows], tab_sh.at[my_rows], stage_sem)

        def zero_acc():
            @pl.loop(0, ROWS_PER_S // CH)
            def _(z):
                pltpu.sync_copy(zb_v, acc_sh.at[pl.ds(s * ROWS_PER_S + z * CH, CH)])

        def fire_gather(jj, b):
            pltpu.async_copy(tab_sh.at[src_v.at[jj]], rows_v.at[b], gsems[b])

        def wait_gather(jj, b):
            pltpu.make_async_copy(tab_sh.at[src_v.at[jj]], rows_v.at[b], gsems[b]).wait()

        def scatter_add(jj, b):
            pltpu.sync_copy(rows_v.at[b], acc_sh.at[dst_v.at[jj]], add=True)

        h_stage = fire_stage(0)
        zero_acc()
        h_stage.wait()
        h_src.wait()
        h_dst.wait()

        for h in range(nh):
            plsc.subcore_barrier()

            for i in range(NB):
                fire_gather(i, i)

            @pl.loop(0, K - NB, step=NB)
            def _(j):
                for i in range(NB):
                    jj = j + i
                    wait_gather(jj, i)
                    scatter_add(jj, i)
                    fire_gather(jj + NB, i)

            for i in range(NB):
                jj = K - NB + i
                wait_gather(jj, i)
                scatter_add(jj, i)

            plsc.subcore_barrier()

            h_wo = pltpu.async_copy(acc_sh.at[my_rows], out_hbm.at[c, h, my_rows], wo_sem)
            if h + 1 < nh:
                h_stage = fire_stage(h + 1)
                h_wo.wait()
                zero_acc()
                h_stage.wait()
            else:
                h_wo.wait()

    return k(*hws, srcr, dstr)


_DOT = functools.partial(
    lax.dot_general,
    dimension_numbers=(((1,), (0,)), ((), ())),
    preferred_element_type=jnp.float32,
)

_R = 1024

_HALF_OUT = (
    [
        pl.BlockSpec((_R, HD), lambda i: (i, 0)),
        pl.BlockSpec((_R, HD), lambda i: (i, 0)),
    ],
    [
        jax.ShapeDtypeStruct((NP, HD), jnp.float32),
        jax.ShapeDtypeStruct((NP, HD), jnp.float32),
    ],
)


def _matmul_split(x, w):
    def body(x_ref, w_ref, o0_ref, o1_ref):
        xx = x_ref[...]
        o0_ref[...] = _DOT(xx, w_ref[:, :HD])
        o1_ref[...] = _DOT(xx, w_ref[:, HD:])

    return pl.pallas_call(
        body,
        grid=(NP // _R,),
        in_specs=[
            pl.BlockSpec((_R, 128), lambda i: (i, 0)),
            pl.BlockSpec((128, 128), lambda i: (0, 0)),
        ],
        out_specs=_HALF_OUT[0],
        out_shape=_HALF_OUT[1],
    )(x, w)


def _relu_matmul_split(p, b, w):
    def body(p_ref, b_ref, w_ref, o0_ref, o1_ref):
        a = jnp.concatenate(
            [
                jnp.maximum(p_ref[0, 0] + p_ref[1, 0] + b_ref[:, :HD], 0.0),
                jnp.maximum(p_ref[0, 1] + p_ref[1, 1] + b_ref[:, HD:], 0.0),
            ],
            axis=-1,
        )
        o0_ref[...] = _DOT(a, w_ref[:, :HD])
        o1_ref[...] = _DOT(a, w_ref[:, HD:])

    return pl.pallas_call(
        body,
        grid=(NP // _R,),
        in_specs=[
            pl.BlockSpec((NC, 2, _R, HD), lambda i: (0, 0, i, 0)),
            pl.BlockSpec((1, 128), lambda i: (0, 0)),
            pl.BlockSpec((128, 128), lambda i: (0, 0)),
        ],
        out_specs=_HALF_OUT[0],
        out_shape=_HALF_OUT[1],
    )(p, b, w)


def _relu_matmul_w3(p, b, w):
    def body(p_ref, b_ref, w_ref, o_ref):
        a = jnp.concatenate(
            [
                jnp.maximum(p_ref[0, 0] + p_ref[1, 0] + b_ref[:, :HD], 0.0),
                jnp.maximum(p_ref[0, 1] + p_ref[1, 1] + b_ref[:, HD:], 0.0),
            ],
            axis=-1,
        )
        o_ref[...] = _DOT(a, w_ref[...])

    return pl.pallas_call(
        body,
        grid=(NP // _R,),
        in_specs=[
            pl.BlockSpec((NC, 2, _R, HD), lambda i: (0, 0, i, 0)),
            pl.BlockSpec((1, 128), lambda i: (0, 0)),
            pl.BlockSpec((128, HD), lambda i: (0, 0)),
        ],
        out_specs=pl.BlockSpec((_R, HD), lambda i: (i, 0)),
        out_shape=jax.ShapeDtypeStruct((NP, HD), jnp.float32),
    )(p, b, w)


def _bias_log_softmax(q, b):
    def body(q_ref, b_ref, o_ref):
        t = q_ref[0, 0] + q_ref[1, 0] + b_ref[...]
        m = jnp.max(t, axis=-1, keepdims=True)
        e = jnp.exp(t - m)
        ssum = jnp.sum(e, axis=-1, keepdims=True)
        o_ref[...] = t - m - jnp.log(ssum)

    return pl.pallas_call(
        body,
        grid=(NP // _R,),
        in_specs=[
            pl.BlockSpec((NC, 1, _R, HD), lambda i: (0, 0, i, 0)),
            pl.BlockSpec((1, HD), lambda i: (0, 0)),
        ],
        out_specs=pl.BlockSpec((_R, HD), lambda i: (i, 0)),
        out_shape=jax.ShapeDtypeStruct((NP, HD), jnp.float32),
    )(q, b)


def kernel(x, edge_index, W1, b1, W2, b2, W3, b3):
    src = edge_index[0].astype(jnp.int32)
    dst = edge_index[1].astype(jnp.int32)
    npad = E_PAD - N_EDGES
    pad_iota = jnp.arange(npad, dtype=jnp.int32)
    srcr = jnp.concatenate([src, pad_iota % N_NODES]).reshape(NW, K, CH)
    dstr = jnp.concatenate([dst, N_NODES + pad_iota % (NP - N_NODES)]).reshape(NW, K, CH)

    xp = jnp.pad(x, ((0, NP - N_NODES), (0, 0)))
    b1r = b1.reshape(1, -1)
    b2r = b2.reshape(1, -1)
    b3r = b3.reshape(1, -1)

    h0, h1 = _matmul_split(xp, W1)
    p = _aggregate((h0, h1), srcr, dstr)
    h0, h1 = _relu_matmul_split(p, b1r, W2)
    p = _aggregate((h0, h1), srcr, dstr)
    h3 = _relu_matmul_w3(p, b2r, W3)
    q = _aggregate((h3,), srcr, dstr)
    out = _bias_log_softmax(q, b3r)
    return out[:N_NODES]
